# Initial kernel scaffold; baseline (speedup 1.0000x reference)
import jax
import jax.numpy as jnp
from jax.experimental import pallas as pl


def kernel(vertex_attr, edgeij_pair, edge_attr):
    raise NotImplementedError("write your pallas kernel here")


# speedup vs baseline: 164.1997x; 164.1997x over previous
import functools

import jax
import jax.numpy as jnp
from jax import lax
from jax.experimental import pallas as pl
from jax.experimental.pallas import tpu as pltpu
from jax.experimental.pallas import tpu_sc as plsc

NC = 2
NS = 16
NW = NC * NS
LANES = 16

N_NODES = 100000
N_EDGES = 3200000

CPT = ((N_NODES + NS - 1) // NS + LANES - 1) // LANES * LANES
NPAD = NS * CPT
EPW = N_EDGES // NW
W = 4000
NWIN = EPW // W


def _sc_body(x_hbm, row_hbm, col_hbm, a_hbm, out_hbm,
             col_v, row_v, a_v, xg_v, c_v, z_v, x_sh, acc_sh):
    cid = lax.axis_index("c")
    sid = lax.axis_index("s")
    wid = cid * NS + sid

    def zbody(i, _):
        z_v[pl.ds(pl.multiple_of(i * LANES, LANES), LANES)] = (
            jnp.zeros((LANES,), jnp.float32))
        return 0
    lax.fori_loop(0, CPT // LANES, zbody, 0)

    nbase = sid * CPT
    pltpu.sync_copy(z_v, acc_sh.at[pl.ds(nbase, CPT)])
    pltpu.sync_copy(x_hbm.at[pl.ds(nbase, CPT)], z_v)
    pltpu.sync_copy(z_v, x_sh.at[pl.ds(nbase, CPT)])
    plsc.subcore_barrier()

    def body(w, _):
        ebase = wid * EPW + w * W
        pltpu.sync_copy(col_hbm.at[pl.ds(ebase, W)], col_v)
        pltpu.sync_copy(a_hbm.at[pl.ds(ebase, W)], a_v)
        pltpu.sync_copy(row_hbm.at[pl.ds(ebase, W)], row_v)
        pltpu.sync_copy(x_sh.at[col_v], xg_v)

        def mbody(i, _):
            s = pl.ds(pl.multiple_of(i * LANES, LANES), LANES)
            c_v[s] = a_v[s] * xg_v[s]
            return 0
        lax.fori_loop(0, W // LANES, mbody, 0)

        pltpu.sync_copy(c_v, acc_sh.at[row_v], add=True)
        return 0
    lax.fori_loop(0, NWIN, body, 0)

    plsc.subcore_barrier()
    pltpu.sync_copy(acc_sh.at[pl.ds(nbase, CPT)], z_v)
    pltpu.sync_copy(z_v, out_hbm.at[pl.ds(cid * NPAD + nbase, CPT)])


def _combine_body(p_ref, b_ref, o_ref):
    o_ref[...] = b_ref[...] - p_ref[0] - p_ref[1]


@jax.jit
def kernel(vertex_attr, edgeij_pair, edge_attr):
    n = vertex_attr.shape[0]
    row = edgeij_pair[0].astype(jnp.int32)
    col = edgeij_pair[1].astype(jnp.int32)
    a = edge_attr[:, 0]
    b = vertex_attr[:, 0]
    x = vertex_attr[:, 1]
    x_pad = jnp.pad(x, (0, NPAD - n))

    mesh = plsc.VectorSubcoreMesh(core_axis_name="c", subcore_axis_name="s")
    partials = pl.kernel(
        _sc_body,
        out_type=jax.ShapeDtypeStruct((NC * NPAD,), jnp.float32),
        mesh=mesh,
        scratch_types=[
            pltpu.VMEM((W,), jnp.int32),
            pltpu.VMEM((W,), jnp.int32),
            pltpu.VMEM((W,), jnp.float32),
            pltpu.VMEM((W,), jnp.float32),
            pltpu.VMEM((W,), jnp.float32),
            pltpu.VMEM((CPT,), jnp.float32),
            pltpu.VMEM_SHARED((NPAD,), jnp.float32),
            pltpu.VMEM_SHARED((NPAD,), jnp.float32),
        ],
    )(x_pad, row, col, a)

    p3 = partials.reshape(NC, 8, NPAD // 8)
    b2 = jnp.pad(b, (0, NPAD - n)).reshape(8, NPAD // 8)
    r2 = pl.pallas_call(
        _combine_body,
        out_shape=jax.ShapeDtypeStruct((8, NPAD // 8), jnp.float32),
    )(p3, b2)
    return r2.reshape(NPAD)[:n].reshape(n, 1)


# speedup vs baseline: 301.4633x; 1.8360x over previous
import jax
import jax.numpy as jnp
from jax import lax
from jax.experimental import pallas as pl
from jax.experimental.pallas import tpu as pltpu
from jax.experimental.pallas import tpu_sc as plsc

NC = 2
NS = 16
NW = NC * NS
LANES = 16

N_NODES = 100000
N_EDGES = 3200000

CPT = ((N_NODES + NS - 1) // NS + LANES - 1) // LANES * LANES
NPAD = NS * CPT
EPW = N_EDGES // NW
W = 2000
NWIN = EPW // W
NBUF = 2


def _sc_body(x_hbm, row_hbm, col_hbm, a_hbm, out_hbm, *refs):
    col_v = refs[0:2]
    row_v = refs[2:4]
    a_v = refs[4:6]
    c_v = refs[6:8]
    rc_v = refs[8:10]
    x_v, acc_sh = refs[10], refs[11]
    in_sem = refs[12:14]
    out_sem = refs[14:16]

    cid = lax.axis_index("c")
    sid = lax.axis_index("s")
    wid = cid * NS + sid
    e0 = wid * EPW

    def fire_in(w, b):
        wc = jnp.minimum(w, NWIN - 1)
        eb = e0 + wc * W
        pltpu.async_copy(col_hbm.at[pl.ds(eb, W)], col_v[b], in_sem[b])
        pltpu.async_copy(row_hbm.at[pl.ds(eb, W)], row_v[b], in_sem[b])
        pltpu.async_copy(a_hbm.at[pl.ds(eb, W)], a_v[b], in_sem[b])

    def wait_in(b):
        pltpu.make_async_copy(col_hbm.at[pl.ds(e0, W)], col_v[b], in_sem[b]).wait()
        pltpu.make_async_copy(row_hbm.at[pl.ds(e0, W)], row_v[b], in_sem[b]).wait()
        pltpu.make_async_copy(a_hbm.at[pl.ds(e0, W)], a_v[b], in_sem[b]).wait()

    def compute(b):
        def mbody(j, _):
            s = pl.ds(pl.multiple_of(j * LANES, LANES), LANES)
            xg = plsc.load_gather(x_v, [col_v[b][s]])
            c_v[b][s] = a_v[b][s] * xg
            rc_v[b][s] = row_v[b][s]
            return 0
        lax.fori_loop(0, W // LANES, mbody, 0)

    def fire_scatter(b):
        pltpu.async_copy(c_v[b], acc_sh.at[rc_v[b]], out_sem[b], add=True)

    def wait_scatter(b):
        pltpu.make_async_copy(c_v[b], acc_sh.at[rc_v[b]], out_sem[b]).wait()

    def zbody(i, _):
        c_v[0][pl.ds(pl.multiple_of(i * LANES, LANES), LANES)] = (
            jnp.zeros((LANES,), jnp.float32))
        return 0
    lax.fori_loop(0, W // LANES, zbody, 0)

    nbase = sid * CPT
    for off in range(0, CPT, W):
        sz = min(W, CPT - off)
        pltpu.sync_copy(c_v[0].at[pl.ds(0, sz)],
                        acc_sh.at[pl.ds(nbase + off, sz)])
    pltpu.sync_copy(x_hbm, x_v)
    for b in range(NBUF):
        fire_in(b, b)
    plsc.subcore_barrier()

    for w in range(NBUF):
        wait_in(w)
        compute(w)
        fire_scatter(w)
        fire_in(w + NBUF, w)

    def body(i, _):
        for b in range(NBUF):
            w = i * NBUF + b
            wait_in(b)
            wait_scatter(b)
            compute(b)
            fire_scatter(b)
            fire_in(w + NBUF, b)
        return 0
    lax.fori_loop(1, NWIN // NBUF, body, 0)

    for b in range(NBUF):
        wait_in(b)
        wait_scatter(b)

    plsc.subcore_barrier()
    for off in range(0, CPT, W):
        sz = min(W, CPT - off)
        pltpu.sync_copy(acc_sh.at[pl.ds(nbase + off, sz)],
                        c_v[0].at[pl.ds(0, sz)])
        pltpu.sync_copy(c_v[0].at[pl.ds(0, sz)],
                        out_hbm.at[pl.ds(cid * NPAD + nbase + off, sz)])


def _combine_body(p_ref, b_ref, o_ref):
    o_ref[...] = b_ref[...] - p_ref[0] - p_ref[1]


@jax.jit
def kernel(vertex_attr, edgeij_pair, edge_attr):
    n = vertex_attr.shape[0]
    row = edgeij_pair[0].astype(jnp.int32)
    col = edgeij_pair[1].astype(jnp.int32)
    a = edge_attr[:, 0]
    b = vertex_attr[:, 0]
    x = vertex_attr[:, 1]

    mesh = plsc.VectorSubcoreMesh(core_axis_name="c", subcore_axis_name="s")
    partials = pl.kernel(
        _sc_body,
        out_type=jax.ShapeDtypeStruct((NC * NPAD,), jnp.float32),
        mesh=mesh,
        compiler_params=pltpu.CompilerParams(needs_layout_passes=False),
        scratch_types=(
            [pltpu.VMEM((W,), jnp.int32) for _ in range(2)]
            + [pltpu.VMEM((W,), jnp.int32) for _ in range(2)]
            + [pltpu.VMEM((W,), jnp.float32) for _ in range(2)]
            + [pltpu.VMEM((W,), jnp.float32) for _ in range(2)]
            + [pltpu.VMEM((W,), jnp.int32) for _ in range(2)]
            + [
                pltpu.VMEM((N_NODES,), jnp.float32),
                pltpu.VMEM_SHARED((NPAD,), jnp.float32),
            ]
            + [pltpu.SemaphoreType.DMA for _ in range(4)]
        ),
    )(x, row, col, a)

    p3 = partials.reshape(NC, 8, NPAD // 8)
    b2 = jnp.pad(b, (0, NPAD - n)).reshape(8, NPAD // 8)
    r2 = pl.pallas_call(
        _combine_body,
        out_shape=jax.ShapeDtypeStruct((8, NPAD // 8), jnp.float32),
    )(p3, b2)
    return r2.reshape(NPAD)[:n].reshape(n, 1)


# speedup vs baseline: 331.3617x; 1.0992x over previous
import jax
import jax.numpy as jnp
from jax import lax
from jax.experimental import pallas as pl
from jax.experimental.pallas import tpu as pltpu
from jax.experimental.pallas import tpu_sc as plsc

NC = 2
NS = 16
NW = NC * NS
LANES = 16

N_NODES = 100000
N_EDGES = 3200000

CPT = ((N_NODES + NS - 1) // NS + LANES - 1) // LANES * LANES
NPAD = NS * CPT
EPW = N_EDGES // NW
W = 2000
NWIN = EPW // W
NBUF = 2


def _sc_body(x_hbm, eij_hbm, a_hbm, out_hbm, *refs):
    col_v = refs[0:2]
    row_v = refs[2:4]
    a_v = refs[4:6]
    c_v = refs[6:8]
    rc_v = refs[8:10]
    x_v, acc_sh = refs[10], refs[11]
    in_sem = refs[12:14]
    out_sem = refs[14:16]

    cid = lax.axis_index("c")
    sid = lax.axis_index("s")
    wid = cid * NS + sid
    e0 = wid * EPW

    def fire_in(w, b):
        wc = jnp.minimum(w, NWIN - 1)
        eb = e0 + wc * W
        pltpu.async_copy(eij_hbm.at[pl.ds(N_EDGES + eb, W)], col_v[b], in_sem[b])
        pltpu.async_copy(eij_hbm.at[pl.ds(eb, W)], row_v[b], in_sem[b])
        pltpu.async_copy(a_hbm.at[pl.ds(eb, W)], a_v[b], in_sem[b])

    def wait_in(b):
        pltpu.make_async_copy(eij_hbm.at[pl.ds(e0, W)], col_v[b], in_sem[b]).wait()
        pltpu.make_async_copy(eij_hbm.at[pl.ds(e0, W)], row_v[b], in_sem[b]).wait()
        pltpu.make_async_copy(a_hbm.at[pl.ds(e0, W)], a_v[b], in_sem[b]).wait()

    def compute(b):
        def mbody(j, _):
            s = pl.ds(pl.multiple_of(j * LANES, LANES), LANES)
            xg = plsc.load_gather(x_v, [col_v[b][s]])
            c_v[b][s] = a_v[b][s] * xg
            rc_v[b][s] = row_v[b][s]
            return 0
        lax.fori_loop(0, W // LANES, mbody, 0)

    def fire_scatter(b):
        pltpu.async_copy(c_v[b], acc_sh.at[rc_v[b]], out_sem[b], add=True)

    def wait_scatter(b):
        pltpu.make_async_copy(c_v[b], acc_sh.at[rc_v[b]], out_sem[b]).wait()

    def zbody(i, _):
        c_v[0][pl.ds(pl.multiple_of(i * LANES, LANES), LANES)] = (
            jnp.zeros((LANES,), jnp.float32))
        return 0
    lax.fori_loop(0, W // LANES, zbody, 0)

    nbase = sid * CPT
    for off in range(0, CPT, W):
        sz = min(W, CPT - off)
        pltpu.sync_copy(c_v[0].at[pl.ds(0, sz)],
                        acc_sh.at[pl.ds(nbase + off, sz)])
    pltpu.sync_copy(x_hbm, x_v)
    for b in range(NBUF):
        fire_in(b, b)
    plsc.subcore_barrier()

    for w in range(NBUF):
        wait_in(w)
        compute(w)
        fire_scatter(w)
        fire_in(w + NBUF, w)

    def body(i, _):
        for b in range(NBUF):
            w = i * NBUF + b
            wait_in(b)
            wait_scatter(b)
            compute(b)
            fire_scatter(b)
            fire_in(w + NBUF, b)
        return 0
    lax.fori_loop(1, NWIN // NBUF, body, 0)

    for b in range(NBUF):
        wait_in(b)
        wait_scatter(b)

    plsc.subcore_barrier()
    for off in range(0, CPT, W):
        sz = min(W, CPT - off)
        pltpu.sync_copy(acc_sh.at[pl.ds(nbase + off, sz)],
                        c_v[0].at[pl.ds(0, sz)])
        pltpu.sync_copy(c_v[0].at[pl.ds(0, sz)],
                        out_hbm.at[pl.ds(cid * NPAD + nbase + off, sz)])


def _combine_body(p_ref, b_ref, o_ref):
    o_ref[...] = (b_ref[...] - p_ref[pl.ds(0, N_NODES)]
                  - p_ref[pl.ds(NPAD, N_NODES)])


@jax.jit
def kernel(vertex_attr, edgeij_pair, edge_attr):
    n = vertex_attr.shape[0]
    eij = edgeij_pair.astype(jnp.int32).reshape(-1)
    a = edge_attr.reshape(-1)
    b = vertex_attr[:, 0]
    x = vertex_attr[:, 1]

    mesh = plsc.VectorSubcoreMesh(core_axis_name="c", subcore_axis_name="s")
    partials = pl.kernel(
        _sc_body,
        out_type=jax.ShapeDtypeStruct((NC * NPAD,), jnp.float32),
        mesh=mesh,
        compiler_params=pltpu.CompilerParams(needs_layout_passes=False),
        scratch_types=(
            [pltpu.VMEM((W,), jnp.int32) for _ in range(2)]
            + [pltpu.VMEM((W,), jnp.int32) for _ in range(2)]
            + [pltpu.VMEM((W,), jnp.float32) for _ in range(2)]
            + [pltpu.VMEM((W,), jnp.float32) for _ in range(2)]
            + [pltpu.VMEM((W,), jnp.int32) for _ in range(2)]
            + [
                pltpu.VMEM((N_NODES,), jnp.float32),
                pltpu.VMEM_SHARED((NPAD,), jnp.float32),
            ]
            + [pltpu.SemaphoreType.DMA for _ in range(4)]
        ),
    )(x, eij, a)

    r = pl.pallas_call(
        _combine_body,
        out_shape=jax.ShapeDtypeStruct((n,), jnp.float32),
    )(partials, b)
    return r.reshape(n, 1)
